# baseline (speedup 1.0000x reference)
import functools

import jax
import jax.numpy as jnp
from jax import lax
from jax.experimental import pallas as pl
from jax.experimental.pallas import tpu as pltpu
from jax.experimental.pallas import tpu_sc as plsc

VOCAB = 1_000_000
EMBED = 64
CTX = 200
NW = 32
BVL = 65536
NG = (VOCAB + BVL - 1) // BVL
REM = VOCAB - (NG - 1) * BVL


def _gather_sum_sc(idx, et):
  mesh = plsc.VectorSubcoreMesh(core_axis_name="c", subcore_axis_name="s")

  @functools.partial(
      pl.kernel,
      mesh=mesh,
      out_type=jax.ShapeDtypeStruct((NW, EMBED), jnp.float32),
      compiler_params=pltpu.CompilerParams(needs_layout_passes=False),
      scratch_types=[
          pltpu.VMEM((CTX + 24,), jnp.int32),
          pltpu.VMEM((8, EMBED, 128), jnp.float32),
          pltpu.VMEM((EMBED,), jnp.float32),
          pltpu.SemaphoreType.DMA,
      ],
  )
  def k(idx_hbm, et_hbm, p_hbm, idx_v, blk_v, acc_v, sem):
    cid = lax.axis_index("c")
    sid = lax.axis_index("s")
    w = sid * 2 + cid
    pltpu.sync_copy(idx_hbm, idx_v.at[pl.ds(0, CTX)])
    zero = jnp.zeros((16,), jnp.float32)
    rows = [lax.iota(jnp.int32, 16) + 16 * g for g in range(4)]
    for g in range(4):
      acc_v[pl.ds(16 * g, 16)] = zero

    @pl.when(w < CTX // 8)
    def _():
      vec = idx_v[pl.ds(8 * w, 16)]
      copies = []
      for e in range(8):
        c = lax.div(vec[e], 128)
        copies.append(pltpu.async_copy(
            et_hbm.at[:, pl.ds(c * 128, 128)], blk_v.at[e], sem))
      acc = [zero, zero, zero, zero]
      for e in range(8):
        copies[e].wait()
        col = jnp.full((16,), lax.rem(vec[e], 128), jnp.int32)
        buf = jnp.full((16,), e, jnp.int32)
        for g in range(4):
          acc[g] = acc[g] + plsc.load_gather(blk_v, [buf, rows[g], col])
      for g in range(4):
        acc_v[pl.ds(16 * g, 16)] = acc[g]

    pltpu.sync_copy(acc_v, p_hbm.at[w])

  return k(idx, et)


def _proj_body(p_ref, wt_ref, b_ref, out_ref, m_ref, acc_ref):
  i = pl.program_id(0)

  @pl.when(i == 0)
  def _():
    m_ref[0] = jnp.float32(-jnp.inf)
    acc_ref[0] = jnp.float32(0.0)

  s = jnp.sum(p_ref[...], axis=0, keepdims=True)
  raw = lax.dot_general(
      s, wt_ref[...],
      dimension_numbers=(((1,), (0,)), ((), ())),
      preferred_element_type=jnp.float32) + b_ref[...].reshape(1, BVL)
  col = i * BVL + lax.broadcasted_iota(jnp.int32, (1, BVL), 1)
  logits = jnp.where(col < VOCAB, raw, jnp.float32(-jnp.inf))

  @pl.when(i < NG - 1)
  def _():
    out_ref[:, pl.ds(i * BVL, BVL)] = logits

  m_old = m_ref[0]
  m_new = jnp.maximum(m_old, jnp.max(logits))
  acc_new = acc_ref[0] * jnp.exp(m_old - m_new) + jnp.sum(
      jnp.exp(logits - m_new))
  m_ref[0] = m_new
  acc_ref[0] = acc_new

  @pl.when(i == NG - 1)
  def _():
    out_ref[:, pl.ds((NG - 1) * BVL, REM)] = logits[:, :REM]
    lse = m_new + jnp.log(acc_new)

    def _sub(j, carry):
      out_ref[:, pl.ds(j * BVL, BVL)] = out_ref[:, pl.ds(j * BVL, BVL)] - lse
      return carry

    lax.fori_loop(0, NG - 1, _sub, 0)
    out_ref[:, pl.ds((NG - 1) * BVL, REM)] = (
        out_ref[:, pl.ds((NG - 1) * BVL, REM)] - lse)


def kernel(inputs, emb_table, W, b):
  idx = inputs.astype(jnp.int32)
  et = emb_table.T
  wt = W.T
  partials = _gather_sum_sc(idx, et)
  out = pl.pallas_call(
      _proj_body,
      grid=(NG,),
      in_specs=[
          pl.BlockSpec((NW, EMBED), lambda i: (0, 0)),
          pl.BlockSpec((EMBED, BVL), lambda i: (0, i)),
          pl.BlockSpec((BVL,), lambda i: (i,)),
      ],
      out_specs=pl.BlockSpec((1, VOCAB), lambda i: (0, 0)),
      out_shape=jax.ShapeDtypeStruct((1, VOCAB), jnp.float32),
      scratch_shapes=[
          pltpu.SMEM((1,), jnp.float32),
          pltpu.SMEM((1,), jnp.float32),
      ],
  )(partials, wt, b)
  return out


# speedup vs baseline: 1.0107x; 1.0107x over previous
import functools

import jax
import jax.numpy as jnp
from jax import lax
from jax.experimental import pallas as pl
from jax.experimental.pallas import tpu as pltpu
from jax.experimental.pallas import tpu_sc as plsc

VOCAB = 1_000_000
EMBED = 64
CTX = 200
NW = 32
BVL = 32768
NG = (VOCAB + BVL - 1) // BVL
REM = VOCAB - (NG - 1) * BVL


def _gather_sum_sc(idx, et):
  mesh = plsc.VectorSubcoreMesh(core_axis_name="c", subcore_axis_name="s")

  @functools.partial(
      pl.kernel,
      mesh=mesh,
      out_type=jax.ShapeDtypeStruct((NW, EMBED), jnp.float32),
      compiler_params=pltpu.CompilerParams(needs_layout_passes=False),
      scratch_types=[
          pltpu.VMEM((CTX + 24,), jnp.int32),
          pltpu.VMEM((8, EMBED, 128), jnp.float32),
          pltpu.VMEM((EMBED,), jnp.float32),
          pltpu.SemaphoreType.DMA,
      ],
  )
  def k(idx_hbm, et_hbm, p_hbm, idx_v, blk_v, acc_v, sem):
    cid = lax.axis_index("c")
    sid = lax.axis_index("s")
    w = sid * 2 + cid
    pltpu.sync_copy(idx_hbm, idx_v.at[pl.ds(0, CTX)])
    zero = jnp.zeros((16,), jnp.float32)
    rows = [lax.iota(jnp.int32, 16) + 16 * g for g in range(4)]
    for g in range(4):
      acc_v[pl.ds(16 * g, 16)] = zero

    @pl.when(w < CTX // 8)
    def _():
      vec = idx_v[pl.ds(8 * w, 16)]
      copies = []
      for e in range(8):
        c = lax.div(vec[e], 128)
        copies.append(pltpu.async_copy(
            et_hbm.at[:, pl.ds(c * 128, 128)], blk_v.at[e], sem))
      acc = [zero, zero, zero, zero]
      for e in range(8):
        copies[e].wait()
        col = jnp.full((16,), lax.rem(vec[e], 128), jnp.int32)
        buf = jnp.full((16,), e, jnp.int32)
        for g in range(4):
          acc[g] = acc[g] + plsc.load_gather(blk_v, [buf, rows[g], col])
      for g in range(4):
        acc_v[pl.ds(16 * g, 16)] = acc[g]

    pltpu.sync_copy(acc_v, p_hbm.at[w])

  return k(idx, et)


def _proj_body(p_ref, wt_ref, b_ref, out_ref, m_ref, acc_ref):
  i = pl.program_id(0)

  @pl.when(i == 0)
  def _():
    m_ref[0] = jnp.float32(-jnp.inf)
    acc_ref[0] = jnp.float32(0.0)

  s = jnp.sum(p_ref[...], axis=0, keepdims=True)
  raw = lax.dot_general(
      s, wt_ref[...],
      dimension_numbers=(((1,), (0,)), ((), ())),
      preferred_element_type=jnp.float32) + b_ref[...].reshape(1, BVL)
  col = i * BVL + lax.broadcasted_iota(jnp.int32, (1, BVL), 1)
  logits = jnp.where(col < VOCAB, raw, jnp.float32(-jnp.inf))

  @pl.when(i < NG - 1)
  def _():
    out_ref[:, pl.ds(i * BVL, BVL)] = logits

  m_old = m_ref[0]
  m_new = jnp.maximum(m_old, jnp.max(logits))
  acc_new = acc_ref[0] * jnp.exp(m_old - m_new) + jnp.sum(
      jnp.exp(logits - m_new))
  m_ref[0] = m_new
  acc_ref[0] = acc_new

  @pl.when(i == NG - 1)
  def _():
    out_ref[:, pl.ds((NG - 1) * BVL, REM)] = logits[:, :REM]
    lse = m_new + jnp.log(acc_new)

    def _sub(j, carry):
      out_ref[:, pl.ds(j * BVL, BVL)] = out_ref[:, pl.ds(j * BVL, BVL)] - lse
      return carry

    lax.fori_loop(0, NG - 1, _sub, 0)
    out_ref[:, pl.ds((NG - 1) * BVL, REM)] = (
        out_ref[:, pl.ds((NG - 1) * BVL, REM)] - lse)


def kernel(inputs, emb_table, W, b):
  idx = inputs.astype(jnp.int32)
  et = emb_table.T
  wt = W.T
  partials = _gather_sum_sc(idx, et)
  out = pl.pallas_call(
      _proj_body,
      grid=(NG,),
      in_specs=[
          pl.BlockSpec((NW, EMBED), lambda i: (0, 0)),
          pl.BlockSpec((EMBED, BVL), lambda i: (0, i)),
          pl.BlockSpec((BVL,), lambda i: (i,)),
      ],
      out_specs=pl.BlockSpec((1, VOCAB), lambda i: (0, 0)),
      out_shape=jax.ShapeDtypeStruct((1, VOCAB), jnp.float32),
      scratch_shapes=[
          pltpu.SMEM((1,), jnp.float32),
          pltpu.SMEM((1,), jnp.float32),
      ],
  )(partials, wt, b)
  return out
